# Initial kernel scaffold; baseline (speedup 1.0000x reference)
import jax
import jax.numpy as jnp
from jax.experimental import pallas as pl


def kernel(input, table):
    raise NotImplementedError("write your pallas kernel here")


# speedup vs baseline: 2.6299x; 2.6299x over previous
import functools

import jax
import jax.numpy as jnp
from jax import lax
from jax.experimental import pallas as pl
from jax.experimental.pallas import tpu as pltpu
from jax.experimental.pallas import tpu_sc as plsc

NC = 2
NS = 16
NW = NC * NS
LANES = 16


def kernel(input, table):
    B, L = input.shape
    _, D = table.shape
    BPW = B // NW
    CH = 2
    NCH = BPW // CH
    KV = D // LANES

    idx = input.reshape(NW, NCH, CH * L).astype(jnp.int32)

    mesh = plsc.VectorSubcoreMesh(core_axis_name="c", subcore_axis_name="s")

    @functools.partial(
        pl.kernel,
        out_type=jax.ShapeDtypeStruct((B, D), jnp.float32),
        mesh=mesh,
        scratch_types=[
            pltpu.VMEM((NCH, CH * L), jnp.int32),
            pltpu.VMEM((CH * L, D), jnp.float32),
            pltpu.VMEM((CH * L, D), jnp.float32),
            pltpu.VMEM((BPW, D), jnp.float32),
            pltpu.SemaphoreType.DMA,
            pltpu.SemaphoreType.DMA,
        ],
        compiler_params=pltpu.CompilerParams(use_tc_tiling_on_sc=False),
    )
    def emb_mean(table_hbm, idx_hbm, out_hbm, idx_v, rows0, rows1, out_v, sem0, sem1):
        wid = lax.axis_index("s") * NC + lax.axis_index("c")
        pltpu.sync_copy(idx_hbm.at[wid], idx_v)

        rows = (rows0, rows1)
        sems = (sem0, sem1)
        inv_l = jnp.float32(1.0 / L)

        pltpu.async_copy(table_hbm.at[idx_v.at[0]], rows0, sem0)
        pltpu.async_copy(table_hbm.at[idx_v.at[1]], rows1, sem1)

        @pl.loop(0, NCH, step=2)
        def _(j):
            for b in range(2):
                jb = j + b
                pltpu.make_async_copy(
                    table_hbm.at[idx_v.at[jb]], rows[b], sems[b]
                ).wait()

                for c in range(CH):
                    accs = [
                        rows[b][c * L, pl.ds(k * LANES, LANES)] for k in range(KV)
                    ]
                    for r in range(1, L):
                        for k in range(KV):
                            accs[k] = accs[k] + rows[b][
                                c * L + r, pl.ds(k * LANES, LANES)
                            ]
                    for k in range(KV):
                        out_v[jb * CH + c, pl.ds(k * LANES, LANES)] = accs[k] * inv_l

                @pl.when(jb + 2 < NCH)
                def _():
                    pltpu.async_copy(table_hbm.at[idx_v.at[jb + 2]], rows[b], sems[b])

        pltpu.sync_copy(out_v, out_hbm.at[pl.ds(wid * BPW, BPW)])

    return emb_mean(table, idx)
